# baseline (speedup 1.0000x reference)
import functools

import jax
import jax.numpy as jnp
from jax import lax
from jax.experimental import pallas as pl
from jax.experimental.pallas import tpu as pltpu
from jax.experimental.pallas import tpu_sc as plsc

N = 10000
NP = 10240
TIN = 12
CH = 16
EMB = 16
TOPK = 12
ALPHA = 1.5
BETA = 0.2
TMIN = 6
F = CH * TMIN
FP = 128
NEG = -3.0e38
IMAX = 2**31 - 1

R0 = 1024
R1 = 256
R3 = 1024

NW = 32
NPW = NP // NW
CHUNK = 8


def _k0_body(xT_ref, e1_ref, e2_ref, w96_ref, beff_ref, wl1_ref, bl1_ref,
             wl2_ref, bl2_ref, h_ref, g1_ref, g2_ref):
    dn = (((1,), (1,)), ((), ()))
    m1 = jnp.tanh(ALPHA * (lax.dot_general(e1_ref[...], wl1_ref[...], dn,
                                           preferred_element_type=jnp.float32)
                           + bl1_ref[...]))
    m2 = jnp.tanh(ALPHA * (lax.dot_general(e2_ref[...], wl2_ref[...], dn,
                                           preferred_element_type=jnp.float32)
                           + bl2_ref[...]))
    g1_ref[...] = jnp.concatenate([m1, m2], axis=1)
    g2_ref[...] = jnp.concatenate([m2, -m1], axis=1)
    h = lax.dot_general(xT_ref[...], w96_ref[...], dn,
                        preferred_element_type=jnp.float32) + beff_ref[...]
    h_ref[...] = jnp.concatenate(
        [jnp.maximum(h, 0.0), jnp.zeros((h.shape[0], FP - F), jnp.float32)],
        axis=1)


def _run_k0(xT, e1, e2, w96, beff, wl1, bl1, wl2, bl2):
    grid = (NP // R0,)
    return pl.pallas_call(
        _k0_body,
        grid=grid,
        in_specs=[
            pl.BlockSpec((R0, TIN), lambda i: (i, 0)),
            pl.BlockSpec((R0, EMB), lambda i: (i, 0)),
            pl.BlockSpec((R0, EMB), lambda i: (i, 0)),
            pl.BlockSpec((F, TIN), lambda i: (0, 0)),
            pl.BlockSpec((1, F), lambda i: (0, 0)),
            pl.BlockSpec((EMB, EMB), lambda i: (0, 0)),
            pl.BlockSpec((1, EMB), lambda i: (0, 0)),
            pl.BlockSpec((EMB, EMB), lambda i: (0, 0)),
            pl.BlockSpec((1, EMB), lambda i: (0, 0)),
        ],
        out_specs=[
            pl.BlockSpec((R0, FP), lambda i: (i, 0)),
            pl.BlockSpec((R0, 2 * EMB), lambda i: (i, 0)),
            pl.BlockSpec((R0, 2 * EMB), lambda i: (i, 0)),
        ],
        out_shape=[
            jax.ShapeDtypeStruct((NP, FP), jnp.float32),
            jax.ShapeDtypeStruct((NP, 2 * EMB), jnp.float32),
            jax.ShapeDtypeStruct((NP, 2 * EMB), jnp.float32),
        ],
    )(xT, e1, e2, w96, beff, wl1, bl1, wl2, bl2)


def _k1_body(g1_ref, g2_ref, cw_ref, idx_ref):
    dn = (((1,), (1,)), ((), ()))
    a = lax.dot_general(g1_ref[...], g2_ref[...], dn,
                        preferred_element_type=jnp.float32)
    colid = lax.broadcasted_iota(jnp.int32, (R1, NP), 1)
    a = jnp.where(colid >= N, NEG, a)
    vals, idxs = [], []
    for _ in range(TOPK):
        m = jnp.max(a, axis=1, keepdims=True)
        sel = jnp.where(a >= m, colid, IMAX)
        ix = jnp.min(sel, axis=1, keepdims=True)
        vals.append(m)
        idxs.append(ix)
        a = jnp.where(colid == ix, NEG, a)
    w = [jnp.maximum(jnp.tanh(ALPHA * v), 0.0) for v in vals]
    d = 1.0
    for wj in w:
        d = d + wj
    inv = (1.0 - BETA) / d
    rowid = (pl.program_id(0) * R1
             + lax.broadcasted_iota(jnp.int32, (R1, 1), 0))
    zc = jnp.zeros((R1, 1), jnp.float32)
    zi = jnp.zeros((R1, 1), jnp.int32)
    cw_ref[...] = jnp.concatenate([wj * inv for wj in w]
                                  + [inv, zc, zc, zc], axis=1)
    idx_ref[...] = jnp.concatenate(idxs + [rowid, zi, zi, zi], axis=1)


def _run_k1(g1, g2):
    grid = (NP // R1,)
    return pl.pallas_call(
        _k1_body,
        grid=grid,
        in_specs=[
            pl.BlockSpec((R1, 2 * EMB), lambda i: (i, 0)),
            pl.BlockSpec((NP, 2 * EMB), lambda i: (0, 0)),
        ],
        out_specs=[
            pl.BlockSpec((R1, 16), lambda i: (i, 0)),
            pl.BlockSpec((R1, 16), lambda i: (i, 0)),
        ],
        out_shape=[
            jax.ShapeDtypeStruct((NP, 16), jnp.float32),
            jax.ShapeDtypeStruct((NP, 16), jnp.int32),
        ],
    )(g1, g2)


NB = 4


def _hop_body(h_hbm, hh_hbm, idx_hbm, cw_hbm, out_hbm,
              hl, cwl, idxl, r0, r1, r2, r3, outc, s0, s1, s2, s3):
    rows = [r0, r1, r2, r3]
    sems = [s0, s1, s2, s3]
    cid = lax.axis_index("c")
    sid = lax.axis_index("s")
    wid = sid * 2 + cid
    base = wid * NPW
    nchunks = NPW // CHUNK
    pltpu.sync_copy(h_hbm.at[pl.ds(base, NPW)], hl)
    pltpu.sync_copy(cw_hbm.at[pl.ds(wid * nchunks, nchunks)], cwl)
    pltpu.sync_copy(idx_hbm.at[pl.ds(wid * nchunks, nchunks)], idxl)

    for b in range(NB):
        pltpu.async_copy(hh_hbm.at[idxl.at[b]], rows[b], sems[b])

    def group(gi, carry):
        for b in range(NB):
            c = gi * NB + b
            pltpu.make_async_copy(hh_hbm.at[idxl.at[c]], rows[b],
                                  sems[b]).wait()
            for n in range(CHUNK):
                g = c * CHUNK + n
                accs = [BETA * hl[g, pl.ds(f * 16, 16)] for f in range(TMIN)]
                cwv = cwl[c, pl.ds(n * 16, 16)]
                for j in range(16):
                    cj = cwv[j]
                    for f in range(TMIN):
                        accs[f] += cj * rows[b][n * 16 + j, pl.ds(f * 16, 16)]
                for f in range(TMIN):
                    outc[n, pl.ds(f * 16, 16)] = accs[f]
                zv = jnp.zeros((16,), jnp.float32)
                for f in range(TMIN, FP // 16):
                    outc[n, pl.ds(f * 16, 16)] = zv
            pltpu.sync_copy(outc, out_hbm.at[pl.ds(base + c * CHUNK, CHUNK)])

            @pl.when(c + NB < nchunks)
            def _():
                pltpu.async_copy(hh_hbm.at[idxl.at[c + NB]], rows[b], sems[b])
        return carry

    lax.fori_loop(0, nchunks // NB, group, 0)


def _run_hop(h, hh, idx_flat, cw):
    mesh = plsc.VectorSubcoreMesh(core_axis_name="c", subcore_axis_name="s")
    return pl.kernel(
        _hop_body,
        out_type=jax.ShapeDtypeStruct((NP, FP), jnp.float32),
        mesh=mesh,
        compiler_params=pltpu.CompilerParams(use_tc_tiling_on_sc=False),
        scratch_types=[
            pltpu.VMEM((NPW, FP), jnp.float32),
            pltpu.VMEM((NPW // CHUNK, CHUNK * 16), jnp.float32),
            pltpu.VMEM((NPW // CHUNK, CHUNK * 16), jnp.int32),
            pltpu.VMEM((CHUNK * 16, FP), jnp.float32),
            pltpu.VMEM((CHUNK * 16, FP), jnp.float32),
            pltpu.VMEM((CHUNK * 16, FP), jnp.float32),
            pltpu.VMEM((CHUNK * 16, FP), jnp.float32),
            pltpu.VMEM((CHUNK, FP), jnp.float32),
            pltpu.SemaphoreType.DMA,
            pltpu.SemaphoreType.DMA,
            pltpu.SemaphoreType.DMA,
            pltpu.SemaphoreType.DMA,
        ],
    )(h, hh, idx_flat, cw)


def _k3_body(h_ref, h1_ref, h2_ref, ka_ref, kb_ref, kc_ref, bg_ref,
             kr1_ref, br1_ref, kavg_ref, br2_ref, y_ref):
    z = (jnp.dot(h_ref[...], ka_ref[...], preferred_element_type=jnp.float32)
         + jnp.dot(h1_ref[...], kb_ref[...], preferred_element_type=jnp.float32)
         + jnp.dot(h2_ref[...], kc_ref[...], preferred_element_type=jnp.float32)
         + bg_ref[...])
    z = jnp.maximum(z, 0.0)
    z = jnp.maximum(jnp.dot(z, kr1_ref[...], preferred_element_type=jnp.float32)
                    + br1_ref[...], 0.0)
    y_ref[...] = (jnp.dot(z, kavg_ref[...], preferred_element_type=jnp.float32)
                  + br2_ref[...])


def _run_k3(h, hh1, hh2, ka, kb, kc, bg_t, kr1, br1_t, kavg, br2_row):
    grid = (NP // R3,)
    full = lambda shape: pl.BlockSpec(shape, lambda i: tuple(0 for _ in shape))
    return pl.pallas_call(
        _k3_body,
        grid=grid,
        in_specs=[
            pl.BlockSpec((R3, FP), lambda i: (i, 0)),
            pl.BlockSpec((R3, FP), lambda i: (i, 0)),
            pl.BlockSpec((R3, FP), lambda i: (i, 0)),
            full((FP, FP)), full((FP, FP)), full((FP, FP)), full((1, FP)),
            full((FP, FP)), full((1, FP)), full((FP, TIN)), full((1, TIN)),
        ],
        out_specs=pl.BlockSpec((R3, TIN), lambda i: (i, 0)),
        out_shape=jax.ShapeDtypeStruct((NP, TIN), jnp.float32),
    )(h, hh1, hh2, ka, kb, kc, bg_t, kr1, br1_t, kavg, br2_row)


def kernel(x, Win, b_in, Wt0, bt0, Wt1, bt1, Wt2, bt2, Wt3, bt3,
           emb1, emb2, Wl1, bl1, Wl2, bl2, Wg, bg, Wr1, br1, Wr2, br2):
    f32 = jnp.float32
    win = Win[:, 0, 0, 0]
    wts = [(Wt0, bt0, 2), (Wt1, bt1, 3), (Wt2, bt2, 6), (Wt3, bt3, 7)]
    w2s, beffc = [], []
    for Wt, bt, k in wts:
        w2s.append((jnp.einsum('cid,i->cd', Wt[:, :, 0, :], win), k))
        beffc.append(bt + jnp.einsum('cid,i->c', Wt[:, :, 0, :], b_in))
    rows = []
    for to in range(TMIN):
        blocks = []
        for w2, k in w2s:
            off = (TIN - 5) - k + to
            blocks.append(jnp.pad(w2, ((0, 0), (off, TIN - off - k))))
        rows.append(jnp.concatenate(blocks, axis=0))
    w96 = jnp.concatenate(rows, axis=0)
    beff = jnp.tile(jnp.concatenate(beffc), TMIN)[None, :]

    eye6 = jnp.eye(TMIN, dtype=f32)
    padw = ((0, FP - F), (0, FP - F))
    Wg2 = Wg[:, :, 0, 0]
    ka = jnp.pad(jnp.kron(eye6, Wg2[:, 0:16].T), padw)
    kb = jnp.pad(jnp.kron(eye6, Wg2[:, 16:32].T), padw)
    kc = jnp.pad(jnp.kron(eye6, Wg2[:, 32:48].T), padw)
    bg_t = jnp.pad(jnp.tile(bg, TMIN), (0, FP - F))[None, :]
    kr1 = jnp.pad(jnp.kron(eye6, Wr1[:, :, 0, 0].T), padw)
    br1_t = jnp.pad(jnp.tile(br1, TMIN), (0, FP - F))[None, :]
    kavg = jnp.pad(jnp.kron(jnp.ones((TMIN, 1), f32), Wr2[:, :, 0, 0].T) / TMIN,
                   ((0, FP - F), (0, 0)))
    br2_row = br2[None, :]

    pad = ((0, NP - N), (0, 0))
    xT = jnp.pad(x[0, :, :, 0].T, pad)
    e1 = jnp.pad(emb1, pad)
    e2 = jnp.pad(emb2, pad)

    h, g1, g2 = _run_k0(xT, e1, e2, w96, beff, Wl1, bl1[None, :],
                        Wl2, bl2[None, :])
    cw, idx = _run_k1(g1, g2)
    idx_flat = idx.reshape(NP // CHUNK, CHUNK * 16)
    cw = cw.reshape(NP // CHUNK, CHUNK * 16)
    hh1 = _run_hop(h, h, idx_flat, cw)
    hh2 = _run_hop(h, hh1, idx_flat, cw)
    y = _run_k3(h, hh1, hh2, ka, kb, kc, bg_t, kr1, br1_t, kavg, br2_row)
    return y[:N].T[None]


# speedup vs baseline: 2.5458x; 2.5458x over previous
import functools

import jax
import jax.numpy as jnp
from jax import lax
from jax.experimental import pallas as pl
from jax.experimental.pallas import tpu as pltpu
from jax.experimental.pallas import tpu_sc as plsc

N = 10000
NP = 10240
TIN = 12
CH = 16
EMB = 16
TOPK = 12
ALPHA = 1.5
BETA = 0.2
TMIN = 6
F = CH * TMIN
FP = 128
NEG = -3.0e38
IMAX = 2**31 - 1

R0 = 1024
R1 = 256
R3 = 1024

NW = 32
NPW = NP // NW
CHUNK = 8


def _k0_body(xT_ref, e1_ref, e2_ref, w96_ref, beff_ref, wl1_ref, bl1_ref,
             wl2_ref, bl2_ref, h_ref, g1_ref, g2_ref):
    dn = (((1,), (1,)), ((), ()))
    m1 = jnp.tanh(ALPHA * (lax.dot_general(e1_ref[...], wl1_ref[...], dn,
                                           preferred_element_type=jnp.float32)
                           + bl1_ref[...]))
    m2 = jnp.tanh(ALPHA * (lax.dot_general(e2_ref[...], wl2_ref[...], dn,
                                           preferred_element_type=jnp.float32)
                           + bl2_ref[...]))
    g1_ref[...] = jnp.concatenate([m1, m2], axis=1)
    g2_ref[...] = jnp.concatenate([m2, -m1], axis=1)
    h = lax.dot_general(xT_ref[...], w96_ref[...], dn,
                        preferred_element_type=jnp.float32) + beff_ref[...]
    h_ref[...] = jnp.concatenate(
        [jnp.maximum(h, 0.0), jnp.zeros((h.shape[0], FP - F), jnp.float32)],
        axis=1)


def _run_k0(xT, e1, e2, w96, beff, wl1, bl1, wl2, bl2):
    grid = (NP // R0,)
    return pl.pallas_call(
        _k0_body,
        grid=grid,
        in_specs=[
            pl.BlockSpec((R0, TIN), lambda i: (i, 0)),
            pl.BlockSpec((R0, EMB), lambda i: (i, 0)),
            pl.BlockSpec((R0, EMB), lambda i: (i, 0)),
            pl.BlockSpec((F, TIN), lambda i: (0, 0)),
            pl.BlockSpec((1, F), lambda i: (0, 0)),
            pl.BlockSpec((EMB, EMB), lambda i: (0, 0)),
            pl.BlockSpec((1, EMB), lambda i: (0, 0)),
            pl.BlockSpec((EMB, EMB), lambda i: (0, 0)),
            pl.BlockSpec((1, EMB), lambda i: (0, 0)),
        ],
        out_specs=[
            pl.BlockSpec((R0, FP), lambda i: (i, 0)),
            pl.BlockSpec((R0, 2 * EMB), lambda i: (i, 0)),
            pl.BlockSpec((R0, 2 * EMB), lambda i: (i, 0)),
        ],
        out_shape=[
            jax.ShapeDtypeStruct((NP, FP), jnp.float32),
            jax.ShapeDtypeStruct((NP, 2 * EMB), jnp.float32),
            jax.ShapeDtypeStruct((NP, 2 * EMB), jnp.float32),
        ],
    )(xT, e1, e2, w96, beff, wl1, bl1, wl2, bl2)


def _k1_body(g1_ref, g2_ref, cw_ref, idx_ref):
    dn = (((1,), (1,)), ((), ()))
    a = lax.dot_general(g1_ref[...], g2_ref[...], dn,
                        preferred_element_type=jnp.float32)
    colid = lax.broadcasted_iota(jnp.int32, (R1, NP), 1)
    a = jnp.where(colid >= N, NEG, a)
    vals, idxs = [], []
    for _ in range(TOPK):
        m = jnp.max(a, axis=1, keepdims=True)
        sel = jnp.where(a >= m, colid, IMAX)
        ix = jnp.min(sel, axis=1, keepdims=True)
        vals.append(m)
        idxs.append(ix)
        a = jnp.where(colid == ix, NEG, a)
    w = [jnp.maximum(jnp.tanh(ALPHA * v), 0.0) for v in vals]
    d = 1.0
    for wj in w:
        d = d + wj
    inv = (1.0 - BETA) / d
    rowid = (pl.program_id(0) * R1
             + lax.broadcasted_iota(jnp.int32, (R1, 1), 0))
    zc = jnp.zeros((R1, 1), jnp.float32)
    zi = jnp.zeros((R1, 1), jnp.int32)
    cw_ref[...] = jnp.concatenate([wj * inv for wj in w]
                                  + [inv + BETA, zc, zc, zc], axis=1)
    idx_ref[...] = jnp.concatenate(idxs + [rowid, zi, zi, zi], axis=1)


def _run_k1(g1, g2):
    grid = (NP // R1,)
    return pl.pallas_call(
        _k1_body,
        grid=grid,
        in_specs=[
            pl.BlockSpec((R1, 2 * EMB), lambda i: (i, 0)),
            pl.BlockSpec((NP, 2 * EMB), lambda i: (0, 0)),
        ],
        out_specs=[
            pl.BlockSpec((R1, 16), lambda i: (i, 0)),
            pl.BlockSpec((R1, 16), lambda i: (i, 0)),
        ],
        out_shape=[
            jax.ShapeDtypeStruct((NP, 16), jnp.float32),
            jax.ShapeDtypeStruct((NP, 16), jnp.int32),
        ],
    )(g1, g2)


NB = 2


def _hop_body(hh_hbm, idx_hbm, cw_hbm, out_hbm,
              sh, cwl, idxl, r0, r1, outc, s0, s1):
    rows = [r0, r1]
    sems = [s0, s1]
    cid = lax.axis_index("c")
    sid = lax.axis_index("s")
    wid = sid * 2 + cid
    base = wid * NPW
    nchunks = NPW // CHUNK
    pltpu.sync_copy(cw_hbm.at[pl.ds(wid * nchunks, nchunks)], cwl)
    pltpu.sync_copy(idx_hbm.at[pl.ds(wid * nchunks, nchunks)], idxl)

    @pl.when(sid == 0)
    def _():
        pltpu.sync_copy(hh_hbm, sh)

    plsc.subcore_barrier()

    for b in range(NB):
        pltpu.async_copy(sh.at[idxl.at[b]], rows[b], sems[b])

    def group(gi, carry):
        for b in range(NB):
            c = gi * NB + b
            pltpu.make_async_copy(sh.at[idxl.at[c]], rows[b], sems[b]).wait()
            for n in range(CHUNK):
                cwv = cwl[c, pl.ds(n * 16, 16)]
                accs = [cwv[0] * rows[b][n * 16, pl.ds(f * 16, 16)]
                        for f in range(TMIN)]
                for j in range(1, 16):
                    cj = cwv[j]
                    for f in range(TMIN):
                        accs[f] += cj * rows[b][n * 16 + j, pl.ds(f * 16, 16)]
                for f in range(TMIN):
                    outc[n, pl.ds(f * 16, 16)] = accs[f]
                zv = jnp.zeros((16,), jnp.float32)
                for f in range(TMIN, FP // 16):
                    outc[n, pl.ds(f * 16, 16)] = zv
            pltpu.sync_copy(outc, out_hbm.at[pl.ds(base + c * CHUNK, CHUNK)])

            @pl.when(c + NB < nchunks)
            def _():
                pltpu.async_copy(sh.at[idxl.at[c + NB]], rows[b], sems[b])
        return carry

    lax.fori_loop(0, nchunks // NB, group, 0)


def _run_hop(hh, idx_flat, cw):
    mesh = plsc.VectorSubcoreMesh(core_axis_name="c", subcore_axis_name="s")
    return pl.kernel(
        _hop_body,
        out_type=jax.ShapeDtypeStruct((NP, FP), jnp.float32),
        mesh=mesh,
        compiler_params=pltpu.CompilerParams(use_tc_tiling_on_sc=False),
        scratch_types=[
            pltpu.VMEM_SHARED((NP, FP), jnp.float32),
            pltpu.VMEM((NPW // CHUNK, CHUNK * 16), jnp.float32),
            pltpu.VMEM((NPW // CHUNK, CHUNK * 16), jnp.int32),
            pltpu.VMEM((CHUNK * 16, FP), jnp.float32),
            pltpu.VMEM((CHUNK * 16, FP), jnp.float32),
            pltpu.VMEM((CHUNK, FP), jnp.float32),
            pltpu.SemaphoreType.DMA,
            pltpu.SemaphoreType.DMA,
        ],
    )(hh, idx_flat, cw)


def _k3_body(h_ref, h1_ref, h2_ref, ka_ref, kb_ref, kc_ref, bg_ref,
             kr1_ref, br1_ref, kavg_ref, br2_ref, y_ref):
    z = (jnp.dot(h_ref[...], ka_ref[...], preferred_element_type=jnp.float32)
         + jnp.dot(h1_ref[...], kb_ref[...], preferred_element_type=jnp.float32)
         + jnp.dot(h2_ref[...], kc_ref[...], preferred_element_type=jnp.float32)
         + bg_ref[...])
    z = jnp.maximum(z, 0.0)
    z = jnp.maximum(jnp.dot(z, kr1_ref[...], preferred_element_type=jnp.float32)
                    + br1_ref[...], 0.0)
    y_ref[...] = (jnp.dot(z, kavg_ref[...], preferred_element_type=jnp.float32)
                  + br2_ref[...])


def _run_k3(h, hh1, hh2, ka, kb, kc, bg_t, kr1, br1_t, kavg, br2_row):
    grid = (NP // R3,)
    full = lambda shape: pl.BlockSpec(shape, lambda i: tuple(0 for _ in shape))
    return pl.pallas_call(
        _k3_body,
        grid=grid,
        in_specs=[
            pl.BlockSpec((R3, FP), lambda i: (i, 0)),
            pl.BlockSpec((R3, FP), lambda i: (i, 0)),
            pl.BlockSpec((R3, FP), lambda i: (i, 0)),
            full((FP, FP)), full((FP, FP)), full((FP, FP)), full((1, FP)),
            full((FP, FP)), full((1, FP)), full((FP, TIN)), full((1, TIN)),
        ],
        out_specs=pl.BlockSpec((R3, TIN), lambda i: (i, 0)),
        out_shape=jax.ShapeDtypeStruct((NP, TIN), jnp.float32),
    )(h, hh1, hh2, ka, kb, kc, bg_t, kr1, br1_t, kavg, br2_row)


def kernel(x, Win, b_in, Wt0, bt0, Wt1, bt1, Wt2, bt2, Wt3, bt3,
           emb1, emb2, Wl1, bl1, Wl2, bl2, Wg, bg, Wr1, br1, Wr2, br2):
    f32 = jnp.float32
    win = Win[:, 0, 0, 0]
    wts = [(Wt0, bt0, 2), (Wt1, bt1, 3), (Wt2, bt2, 6), (Wt3, bt3, 7)]
    w2s, beffc = [], []
    for Wt, bt, k in wts:
        w2s.append((jnp.einsum('cid,i->cd', Wt[:, :, 0, :], win), k))
        beffc.append(bt + jnp.einsum('cid,i->c', Wt[:, :, 0, :], b_in))
    rows = []
    for to in range(TMIN):
        blocks = []
        for w2, k in w2s:
            off = (TIN - 5) - k + to
            blocks.append(jnp.pad(w2, ((0, 0), (off, TIN - off - k))))
        rows.append(jnp.concatenate(blocks, axis=0))
    w96 = jnp.concatenate(rows, axis=0)
    beff = jnp.tile(jnp.concatenate(beffc), TMIN)[None, :]

    eye6 = jnp.eye(TMIN, dtype=f32)
    padw = ((0, FP - F), (0, FP - F))
    Wg2 = Wg[:, :, 0, 0]
    ka0 = jnp.kron(eye6, Wg2[:, 0:16].T)
    kb0 = jnp.kron(eye6, Wg2[:, 16:32].T)
    kc0 = jnp.kron(eye6, Wg2[:, 32:48].T)
    ka = jnp.pad(ka0 + BETA * kc0, padw)
    kb = jnp.pad(kb0 - BETA * kc0, padw)
    kc = jnp.pad(kc0, padw)
    bg_t = jnp.pad(jnp.tile(bg, TMIN), (0, FP - F))[None, :]
    kr1 = jnp.pad(jnp.kron(eye6, Wr1[:, :, 0, 0].T), padw)
    br1_t = jnp.pad(jnp.tile(br1, TMIN), (0, FP - F))[None, :]
    kavg = jnp.pad(jnp.kron(jnp.ones((TMIN, 1), f32), Wr2[:, :, 0, 0].T) / TMIN,
                   ((0, FP - F), (0, 0)))
    br2_row = br2[None, :]

    pad = ((0, NP - N), (0, 0))
    xT = jnp.pad(x[0, :, :, 0].T, pad)
    e1 = jnp.pad(emb1, pad)
    e2 = jnp.pad(emb2, pad)

    h, g1, g2 = _run_k0(xT, e1, e2, w96, beff, Wl1, bl1[None, :],
                        Wl2, bl2[None, :])
    cw, idx = _run_k1(g1, g2)
    idx_flat = idx.reshape(NP // CHUNK, CHUNK * 16)
    cw = cw.reshape(NP // CHUNK, CHUNK * 16)
    t1 = _run_hop(h, idx_flat, cw)
    t2 = _run_hop(t1, idx_flat, cw)
    y = _run_k3(h, t1, t2, ka, kb, kc, bg_t, kr1, br1_t, kavg, br2_row)
    return y[:N].T[None]


# speedup vs baseline: 2.8927x; 1.1363x over previous
import functools

import jax
import jax.numpy as jnp
from jax import lax
from jax.experimental import pallas as pl
from jax.experimental.pallas import tpu as pltpu
from jax.experimental.pallas import tpu_sc as plsc

N = 10000
NP = 10240
TIN = 12
CH = 16
EMB = 16
TOPK = 12
ALPHA = 1.5
BETA = 0.2
TMIN = 6
F = CH * TMIN
FP = 128
NEG = -3.0e38
IMAX = 2**31 - 1

R0 = 1024
R1 = 256
R3 = 1024

NW = 32
NPW = NP // NW
CHUNK = 8


def _k0_body(xT_ref, e1_ref, e2_ref, w96_ref, beff_ref, wl1_ref, bl1_ref,
             wl2_ref, bl2_ref, h_ref, g1_ref, g2_ref):
    dn = (((1,), (1,)), ((), ()))
    m1 = jnp.tanh(ALPHA * (lax.dot_general(e1_ref[...], wl1_ref[...], dn,
                                           preferred_element_type=jnp.float32)
                           + bl1_ref[...]))
    m2 = jnp.tanh(ALPHA * (lax.dot_general(e2_ref[...], wl2_ref[...], dn,
                                           preferred_element_type=jnp.float32)
                           + bl2_ref[...]))
    g1_ref[...] = jnp.concatenate([m1, m2], axis=1)
    g2_ref[...] = jnp.concatenate([m2, -m1], axis=1)
    h = lax.dot_general(xT_ref[...], w96_ref[...], dn,
                        preferred_element_type=jnp.float32) + beff_ref[...]
    h_ref[...] = jnp.concatenate(
        [jnp.maximum(h, 0.0), jnp.zeros((h.shape[0], FP - F), jnp.float32)],
        axis=1)


def _run_k0(xT, e1, e2, w96, beff, wl1, bl1, wl2, bl2):
    grid = (NP // R0,)
    return pl.pallas_call(
        _k0_body,
        grid=grid,
        in_specs=[
            pl.BlockSpec((R0, TIN), lambda i: (i, 0)),
            pl.BlockSpec((R0, EMB), lambda i: (i, 0)),
            pl.BlockSpec((R0, EMB), lambda i: (i, 0)),
            pl.BlockSpec((F, TIN), lambda i: (0, 0)),
            pl.BlockSpec((1, F), lambda i: (0, 0)),
            pl.BlockSpec((EMB, EMB), lambda i: (0, 0)),
            pl.BlockSpec((1, EMB), lambda i: (0, 0)),
            pl.BlockSpec((EMB, EMB), lambda i: (0, 0)),
            pl.BlockSpec((1, EMB), lambda i: (0, 0)),
        ],
        out_specs=[
            pl.BlockSpec((R0, FP), lambda i: (i, 0)),
            pl.BlockSpec((R0, 2 * EMB), lambda i: (i, 0)),
            pl.BlockSpec((R0, 2 * EMB), lambda i: (i, 0)),
        ],
        out_shape=[
            jax.ShapeDtypeStruct((NP, FP), jnp.float32),
            jax.ShapeDtypeStruct((NP, 2 * EMB), jnp.float32),
            jax.ShapeDtypeStruct((NP, 2 * EMB), jnp.float32),
        ],
    )(xT, e1, e2, w96, beff, wl1, bl1, wl2, bl2)


def _k1_body(g1_ref, g2_ref, cw_ref, idx_ref):
    dn = (((1,), (1,)), ((), ()))
    a = lax.dot_general(g1_ref[...], g2_ref[...], dn,
                        preferred_element_type=jnp.float32)
    colid = lax.broadcasted_iota(jnp.int32, (R1, NP), 1)
    a = jnp.where(colid >= N, NEG, a)
    vals, idxs = [], []
    for _ in range(TOPK):
        m = jnp.max(a, axis=1, keepdims=True)
        ge = a >= m
        ix = jnp.min(jnp.where(ge, colid, IMAX), axis=1, keepdims=True)
        vals.append(m)
        idxs.append(ix)
        a = jnp.where(ge, NEG, a)
    w = [jnp.maximum(jnp.tanh(ALPHA * v), 0.0) for v in vals]
    d = 1.0
    for wj in w:
        d = d + wj
    inv = (1.0 - BETA) / d
    rowid = (pl.program_id(0) * R1
             + lax.broadcasted_iota(jnp.int32, (R1, 1), 0))
    zc = jnp.zeros((R1, 1), jnp.float32)
    zi = jnp.zeros((R1, 1), jnp.int32)
    cw_ref[...] = jnp.concatenate([wj * inv for wj in w]
                                  + [inv + BETA, zc, zc, zc], axis=1)
    idx_ref[...] = jnp.concatenate(idxs + [rowid, zi, zi, zi], axis=1)


def _run_k1(g1, g2):
    grid = (NP // R1,)
    return pl.pallas_call(
        _k1_body,
        grid=grid,
        in_specs=[
            pl.BlockSpec((R1, 2 * EMB), lambda i: (i, 0)),
            pl.BlockSpec((NP, 2 * EMB), lambda i: (0, 0)),
        ],
        out_specs=[
            pl.BlockSpec((R1, 16), lambda i: (i, 0)),
            pl.BlockSpec((R1, 16), lambda i: (i, 0)),
        ],
        out_shape=[
            jax.ShapeDtypeStruct((NP, 16), jnp.float32),
            jax.ShapeDtypeStruct((NP, 16), jnp.int32),
        ],
    )(g1, g2)


NB = 2


def _hop_body(hh_hbm, idx_hbm, cw_hbm, out_hbm,
              sh, cwl, idxl, r0, r1, outc, s0, s1):
    rows = [r0, r1]
    sems = [s0, s1]
    cid = lax.axis_index("c")
    sid = lax.axis_index("s")
    wid = sid * 2 + cid
    base = wid * NPW
    nchunks = NPW // CHUNK
    pltpu.sync_copy(cw_hbm.at[pl.ds(wid * nchunks, nchunks)], cwl)
    pltpu.sync_copy(idx_hbm.at[pl.ds(wid * nchunks, nchunks)], idxl)

    @pl.when(sid == 0)
    def _():
        pltpu.sync_copy(hh_hbm, sh)

    plsc.subcore_barrier()

    for b in range(NB):
        pltpu.async_copy(sh.at[idxl.at[b]], rows[b], sems[b])

    def group(gi, carry):
        for b in range(NB):
            c = gi * NB + b
            pltpu.make_async_copy(sh.at[idxl.at[c]], rows[b], sems[b]).wait()
            for n in range(CHUNK):
                cwv = cwl[c, pl.ds(n * 16, 16)]
                accs = [cwv[0] * rows[b][n * 16, pl.ds(f * 16, 16)]
                        for f in range(TMIN)]
                for j in range(1, 16):
                    cj = cwv[j]
                    for f in range(TMIN):
                        accs[f] += cj * rows[b][n * 16 + j, pl.ds(f * 16, 16)]
                for f in range(TMIN):
                    outc[n, pl.ds(f * 16, 16)] = accs[f]
                zv = jnp.zeros((16,), jnp.float32)
                for f in range(TMIN, FP // 16):
                    outc[n, pl.ds(f * 16, 16)] = zv
            pltpu.sync_copy(outc, out_hbm.at[pl.ds(base + c * CHUNK, CHUNK)])

            @pl.when(c + NB < nchunks)
            def _():
                pltpu.async_copy(sh.at[idxl.at[c + NB]], rows[b], sems[b])
        return carry

    lax.fori_loop(0, nchunks // NB, group, 0)


def _run_hop(hh, idx_flat, cw):
    mesh = plsc.VectorSubcoreMesh(core_axis_name="c", subcore_axis_name="s")
    return pl.kernel(
        _hop_body,
        out_type=jax.ShapeDtypeStruct((NP, FP), jnp.float32),
        mesh=mesh,
        scratch_types=[
            pltpu.VMEM_SHARED((NP, FP), jnp.float32),
            pltpu.VMEM((NPW // CHUNK, CHUNK * 16), jnp.float32),
            pltpu.VMEM((NPW // CHUNK, CHUNK * 16), jnp.int32),
            pltpu.VMEM((CHUNK * 16, FP), jnp.float32),
            pltpu.VMEM((CHUNK * 16, FP), jnp.float32),
            pltpu.VMEM((CHUNK, FP), jnp.float32),
            pltpu.SemaphoreType.DMA,
            pltpu.SemaphoreType.DMA,
        ],
    )(hh, idx_flat, cw)


def _k3_body(h_ref, h1_ref, h2_ref, ka_ref, kb_ref, kc_ref, bg_ref,
             kr1_ref, br1_ref, kavg_ref, br2_ref, y_ref):
    z = (jnp.dot(h_ref[...], ka_ref[...], preferred_element_type=jnp.float32)
         + jnp.dot(h1_ref[...], kb_ref[...], preferred_element_type=jnp.float32)
         + jnp.dot(h2_ref[...], kc_ref[...], preferred_element_type=jnp.float32)
         + bg_ref[...])
    z = jnp.maximum(z, 0.0)
    z = jnp.maximum(jnp.dot(z, kr1_ref[...], preferred_element_type=jnp.float32)
                    + br1_ref[...], 0.0)
    y_ref[...] = (jnp.dot(z, kavg_ref[...], preferred_element_type=jnp.float32)
                  + br2_ref[...])


def _run_k3(h, hh1, hh2, ka, kb, kc, bg_t, kr1, br1_t, kavg, br2_row):
    grid = (NP // R3,)
    full = lambda shape: pl.BlockSpec(shape, lambda i: tuple(0 for _ in shape))
    return pl.pallas_call(
        _k3_body,
        grid=grid,
        in_specs=[
            pl.BlockSpec((R3, FP), lambda i: (i, 0)),
            pl.BlockSpec((R3, FP), lambda i: (i, 0)),
            pl.BlockSpec((R3, FP), lambda i: (i, 0)),
            full((FP, FP)), full((FP, FP)), full((FP, FP)), full((1, FP)),
            full((FP, FP)), full((1, FP)), full((FP, TIN)), full((1, TIN)),
        ],
        out_specs=pl.BlockSpec((R3, TIN), lambda i: (i, 0)),
        out_shape=jax.ShapeDtypeStruct((NP, TIN), jnp.float32),
    )(h, hh1, hh2, ka, kb, kc, bg_t, kr1, br1_t, kavg, br2_row)


def kernel(x, Win, b_in, Wt0, bt0, Wt1, bt1, Wt2, bt2, Wt3, bt3,
           emb1, emb2, Wl1, bl1, Wl2, bl2, Wg, bg, Wr1, br1, Wr2, br2):
    f32 = jnp.float32
    win = Win[:, 0, 0, 0]
    wts = [(Wt0, bt0, 2), (Wt1, bt1, 3), (Wt2, bt2, 6), (Wt3, bt3, 7)]
    w2s, beffc = [], []
    for Wt, bt, k in wts:
        w2s.append((jnp.einsum('cid,i->cd', Wt[:, :, 0, :], win), k))
        beffc.append(bt + jnp.einsum('cid,i->c', Wt[:, :, 0, :], b_in))
    rows = []
    for to in range(TMIN):
        blocks = []
        for w2, k in w2s:
            off = (TIN - 5) - k + to
            blocks.append(jnp.pad(w2, ((0, 0), (off, TIN - off - k))))
        rows.append(jnp.concatenate(blocks, axis=0))
    w96 = jnp.concatenate(rows, axis=0)
    beff = jnp.tile(jnp.concatenate(beffc), TMIN)[None, :]

    eye6 = jnp.eye(TMIN, dtype=f32)
    padw = ((0, FP - F), (0, FP - F))
    Wg2 = Wg[:, :, 0, 0]
    ka0 = jnp.kron(eye6, Wg2[:, 0:16].T)
    kb0 = jnp.kron(eye6, Wg2[:, 16:32].T)
    kc0 = jnp.kron(eye6, Wg2[:, 32:48].T)
    ka = jnp.pad(ka0 + BETA * kc0, padw)
    kb = jnp.pad(kb0 - BETA * kc0, padw)
    kc = jnp.pad(kc0, padw)
    bg_t = jnp.pad(jnp.tile(bg, TMIN), (0, FP - F))[None, :]
    kr1 = jnp.pad(jnp.kron(eye6, Wr1[:, :, 0, 0].T), padw)
    br1_t = jnp.pad(jnp.tile(br1, TMIN), (0, FP - F))[None, :]
    kavg = jnp.pad(jnp.kron(jnp.ones((TMIN, 1), f32), Wr2[:, :, 0, 0].T) / TMIN,
                   ((0, FP - F), (0, 0)))
    br2_row = br2[None, :]

    pad = ((0, NP - N), (0, 0))
    xT = jnp.pad(x[0, :, :, 0].T, pad)
    e1 = jnp.pad(emb1, pad)
    e2 = jnp.pad(emb2, pad)

    h, g1, g2 = _run_k0(xT, e1, e2, w96, beff, Wl1, bl1[None, :],
                        Wl2, bl2[None, :])
    cw, idx = _run_k1(g1, g2)
    idx_flat = idx.reshape(NP // CHUNK, CHUNK * 16)
    cw = cw.reshape(NP // CHUNK, CHUNK * 16)
    t1 = _run_hop(h, idx_flat, cw)
    t2 = _run_hop(t1, idx_flat, cw)
    y = _run_k3(h, t1, t2, ka, kb, kc, bg_t, kr1, br1_t, kavg, br2_row)
    return y[:N].T[None]


# speedup vs baseline: 2.9599x; 1.0232x over previous
import functools

import jax
import jax.numpy as jnp
from jax import lax
from jax.experimental import pallas as pl
from jax.experimental.pallas import tpu as pltpu
from jax.experimental.pallas import tpu_sc as plsc

N = 10000
NP = 10240
TIN = 12
CH = 16
EMB = 16
TOPK = 12
ALPHA = 1.5
BETA = 0.2
TMIN = 6
F = CH * TMIN
FP = 128
NEG = -3.0e38
IMAX = 2**31 - 1

R0 = 1024
R1 = 256
R3 = 1024

NW = 32
NPW = NP // NW
CHUNK = 8


def _k0_body(xr_ref, e1_ref, e2_ref, w96_ref, beff_ref, wl1_ref, bl1_ref,
             wl2_ref, bl2_ref, h_ref, g1_ref, g2_ref):
    dn = (((1,), (1,)), ((), ()))
    m1 = jnp.tanh(ALPHA * (lax.dot_general(e1_ref[...], wl1_ref[...], dn,
                                           preferred_element_type=jnp.float32)
                           + bl1_ref[...]))
    m2 = jnp.tanh(ALPHA * (lax.dot_general(e2_ref[...], wl2_ref[...], dn,
                                           preferred_element_type=jnp.float32)
                           + bl2_ref[...]))
    g1_ref[...] = jnp.concatenate([m1, m2], axis=1)
    g2_ref[...] = jnp.concatenate([m2, -m1], axis=1)
    h = lax.dot_general(xr_ref[...], w96_ref[...], (((0,), (1,)), ((), ())),
                        preferred_element_type=jnp.float32) + beff_ref[...]
    h_ref[...] = jnp.concatenate(
        [jnp.maximum(h, 0.0), jnp.zeros((h.shape[0], FP - F), jnp.float32)],
        axis=1)


def _run_k0(xr, e1, e2, w96, beff, wl1, bl1, wl2, bl2):
    grid = (NP // R0,)
    return pl.pallas_call(
        _k0_body,
        grid=grid,
        in_specs=[
            pl.BlockSpec((TIN, R0), lambda i: (0, i)),
            pl.BlockSpec((R0, EMB), lambda i: (i, 0)),
            pl.BlockSpec((R0, EMB), lambda i: (i, 0)),
            pl.BlockSpec((F, TIN), lambda i: (0, 0)),
            pl.BlockSpec((1, F), lambda i: (0, 0)),
            pl.BlockSpec((EMB, EMB), lambda i: (0, 0)),
            pl.BlockSpec((1, EMB), lambda i: (0, 0)),
            pl.BlockSpec((EMB, EMB), lambda i: (0, 0)),
            pl.BlockSpec((1, EMB), lambda i: (0, 0)),
        ],
        out_specs=[
            pl.BlockSpec((R0, FP), lambda i: (i, 0)),
            pl.BlockSpec((R0, 2 * EMB), lambda i: (i, 0)),
            pl.BlockSpec((R0, 2 * EMB), lambda i: (i, 0)),
        ],
        out_shape=[
            jax.ShapeDtypeStruct((NP, FP), jnp.float32),
            jax.ShapeDtypeStruct((NP, 2 * EMB), jnp.float32),
            jax.ShapeDtypeStruct((NP, 2 * EMB), jnp.float32),
        ],
    )(xr, e1, e2, w96, beff, wl1, bl1, wl2, bl2)


def _k1_body(g1_ref, g2_ref, cw_ref, idx_ref):
    dn = (((1,), (1,)), ((), ()))
    a = lax.dot_general(g1_ref[...], g2_ref[...], dn,
                        preferred_element_type=jnp.float32)
    colid = lax.broadcasted_iota(jnp.int32, (R1, NP), 1)
    a = jnp.where(colid >= N, NEG, a)
    vals, idxs = [], []
    for _ in range(TOPK):
        m = jnp.max(a, axis=1, keepdims=True)
        ge = a >= m
        ix = jnp.min(jnp.where(ge, colid, IMAX), axis=1, keepdims=True)
        vals.append(m)
        idxs.append(ix)
        a = jnp.where(ge, NEG, a)
    w = [jnp.maximum(jnp.tanh(ALPHA * v), 0.0) for v in vals]
    d = 1.0
    for wj in w:
        d = d + wj
    inv = (1.0 - BETA) / d
    rowid = (pl.program_id(0) * R1
             + lax.broadcasted_iota(jnp.int32, (R1, 1), 0))
    zc = jnp.zeros((R1, 1), jnp.float32)
    zi = jnp.zeros((R1, 1), jnp.int32)
    cw_ref[...] = jnp.concatenate([wj * inv for wj in w]
                                  + [inv + BETA, zc, zc, zc], axis=1)
    idx_ref[...] = jnp.concatenate(idxs + [rowid, zi, zi, zi], axis=1)


def _run_k1(g1, g2):
    grid = (NP // R1,)
    return pl.pallas_call(
        _k1_body,
        grid=grid,
        in_specs=[
            pl.BlockSpec((R1, 2 * EMB), lambda i: (i, 0)),
            pl.BlockSpec((NP, 2 * EMB), lambda i: (0, 0)),
        ],
        out_specs=[
            pl.BlockSpec((R1, 16), lambda i: (i, 0)),
            pl.BlockSpec((R1, 16), lambda i: (i, 0)),
        ],
        out_shape=[
            jax.ShapeDtypeStruct((NP, 16), jnp.float32),
            jax.ShapeDtypeStruct((NP, 16), jnp.int32),
        ],
    )(g1, g2)


NB = 2


def _hop_body(hh_hbm, idx_hbm, cw_hbm, out_hbm,
              sh, cwl, idxl, r0, r1, outc, s0, s1):
    rows = [r0, r1]
    sems = [s0, s1]
    cid = lax.axis_index("c")
    sid = lax.axis_index("s")
    wid = sid * 2 + cid
    base = wid * NPW
    nchunks = NPW // CHUNK
    pltpu.sync_copy(cw_hbm.at[pl.ds(wid * nchunks, nchunks)], cwl)
    pltpu.sync_copy(idx_hbm.at[pl.ds(wid * nchunks, nchunks)], idxl)

    @pl.when(sid == 0)
    def _():
        pltpu.sync_copy(hh_hbm, sh)

    plsc.subcore_barrier()

    for n in range(CHUNK):
        zv = jnp.zeros((16,), jnp.float32)
        for f in range(TMIN, FP // 16):
            outc[n, pl.ds(f * 16, 16)] = zv

    for b in range(NB):
        pltpu.async_copy(sh.at[idxl.at[b]], rows[b], sems[b])

    def group(gi, carry):
        for b in range(NB):
            c = gi * NB + b
            pltpu.make_async_copy(sh.at[idxl.at[c]], rows[b], sems[b]).wait()
            for n in range(CHUNK):
                cwv = cwl[c, pl.ds(n * 16, 16)]
                accs = [cwv[0] * rows[b][n * 16, pl.ds(f * 16, 16)]
                        for f in range(TMIN)]
                for j in range(1, 13):
                    cj = cwv[j]
                    for f in range(TMIN):
                        accs[f] += cj * rows[b][n * 16 + j, pl.ds(f * 16, 16)]
                for f in range(TMIN):
                    outc[n, pl.ds(f * 16, 16)] = accs[f]
            pltpu.sync_copy(outc, out_hbm.at[pl.ds(base + c * CHUNK, CHUNK)])

            @pl.when(c + NB < nchunks)
            def _():
                pltpu.async_copy(sh.at[idxl.at[c + NB]], rows[b], sems[b])
        return carry

    lax.fori_loop(0, nchunks // NB, group, 0)


def _run_hop(hh, idx_flat, cw):
    mesh = plsc.VectorSubcoreMesh(core_axis_name="c", subcore_axis_name="s")
    return pl.kernel(
        _hop_body,
        out_type=jax.ShapeDtypeStruct((NP, FP), jnp.float32),
        mesh=mesh,
        scratch_types=[
            pltpu.VMEM_SHARED((NP, FP), jnp.float32),
            pltpu.VMEM((NPW // CHUNK, CHUNK * 16), jnp.float32),
            pltpu.VMEM((NPW // CHUNK, CHUNK * 16), jnp.int32),
            pltpu.VMEM((CHUNK * 16, FP), jnp.float32),
            pltpu.VMEM((CHUNK * 16, FP), jnp.float32),
            pltpu.VMEM((CHUNK, FP), jnp.float32),
            pltpu.SemaphoreType.DMA,
            pltpu.SemaphoreType.DMA,
        ],
    )(hh, idx_flat, cw)


def _k3_body(h_ref, h1_ref, h2_ref, ka_ref, kb_ref, kc_ref, bg_ref,
             kr1_ref, br1_ref, kavg_ref, br2_ref, y_ref):
    z = (jnp.dot(h_ref[...], ka_ref[...], preferred_element_type=jnp.float32)
         + jnp.dot(h1_ref[...], kb_ref[...], preferred_element_type=jnp.float32)
         + jnp.dot(h2_ref[...], kc_ref[...], preferred_element_type=jnp.float32)
         + bg_ref[...])
    z = jnp.maximum(z, 0.0)
    z = jnp.maximum(jnp.dot(z, kr1_ref[...], preferred_element_type=jnp.float32)
                    + br1_ref[...], 0.0)
    y = (jnp.dot(z, kavg_ref[...], preferred_element_type=jnp.float32)
         + br2_ref[...])
    y_ref[...] = y.T


def _run_k3(h, hh1, hh2, ka, kb, kc, bg_t, kr1, br1_t, kavg, br2_row):
    grid = (NP // R3,)
    full = lambda shape: pl.BlockSpec(shape, lambda i: tuple(0 for _ in shape))
    return pl.pallas_call(
        _k3_body,
        grid=grid,
        in_specs=[
            pl.BlockSpec((R3, FP), lambda i: (i, 0)),
            pl.BlockSpec((R3, FP), lambda i: (i, 0)),
            pl.BlockSpec((R3, FP), lambda i: (i, 0)),
            full((FP, FP)), full((FP, FP)), full((FP, FP)), full((1, FP)),
            full((FP, FP)), full((1, FP)), full((FP, TIN)), full((1, TIN)),
        ],
        out_specs=pl.BlockSpec((TIN, R3), lambda i: (0, i)),
        out_shape=jax.ShapeDtypeStruct((TIN, NP), jnp.float32),
    )(h, hh1, hh2, ka, kb, kc, bg_t, kr1, br1_t, kavg, br2_row)


def kernel(x, Win, b_in, Wt0, bt0, Wt1, bt1, Wt2, bt2, Wt3, bt3,
           emb1, emb2, Wl1, bl1, Wl2, bl2, Wg, bg, Wr1, br1, Wr2, br2):
    f32 = jnp.float32
    win = Win[:, 0, 0, 0]
    wts = [(Wt0, bt0, 2), (Wt1, bt1, 3), (Wt2, bt2, 6), (Wt3, bt3, 7)]
    w2s, beffc = [], []
    for Wt, bt, k in wts:
        w2s.append((jnp.einsum('cid,i->cd', Wt[:, :, 0, :], win), k))
        beffc.append(bt + jnp.einsum('cid,i->c', Wt[:, :, 0, :], b_in))
    rows = []
    for to in range(TMIN):
        blocks = []
        for w2, k in w2s:
            off = (TIN - 5) - k + to
            blocks.append(jnp.pad(w2, ((0, 0), (off, TIN - off - k))))
        rows.append(jnp.concatenate(blocks, axis=0))
    w96 = jnp.concatenate(rows, axis=0)
    beff = jnp.tile(jnp.concatenate(beffc), TMIN)[None, :]

    eye6 = jnp.eye(TMIN, dtype=f32)
    padw = ((0, FP - F), (0, FP - F))
    Wg2 = Wg[:, :, 0, 0]
    ka0 = jnp.kron(eye6, Wg2[:, 0:16].T)
    kb0 = jnp.kron(eye6, Wg2[:, 16:32].T)
    kc0 = jnp.kron(eye6, Wg2[:, 32:48].T)
    ka = jnp.pad(ka0 + BETA * kc0, padw)
    kb = jnp.pad(kb0 - BETA * kc0, padw)
    kc = jnp.pad(kc0, padw)
    bg_t = jnp.pad(jnp.tile(bg, TMIN), (0, FP - F))[None, :]
    kr1 = jnp.pad(jnp.kron(eye6, Wr1[:, :, 0, 0].T), padw)
    br1_t = jnp.pad(jnp.tile(br1, TMIN), (0, FP - F))[None, :]
    kavg = jnp.pad(jnp.kron(jnp.ones((TMIN, 1), f32), Wr2[:, :, 0, 0].T) / TMIN,
                   ((0, FP - F), (0, 0)))
    br2_row = br2[None, :]

    pad = ((0, NP - N), (0, 0))
    xr = jnp.pad(x[0, :, :, 0], ((0, 0), (0, NP - N)))
    e1 = jnp.pad(emb1, pad)
    e2 = jnp.pad(emb2, pad)

    h, g1, g2 = _run_k0(xr, e1, e2, w96, beff, Wl1, bl1[None, :],
                        Wl2, bl2[None, :])
    cw, idx = _run_k1(g1, g2)
    idx_flat = idx.reshape(NP // CHUNK, CHUNK * 16)
    cw = cw.reshape(NP // CHUNK, CHUNK * 16)
    t1 = _run_hop(h, idx_flat, cw)
    t2 = _run_hop(t1, idx_flat, cw)
    y = _run_k3(h, t1, t2, ka, kb, kc, bg_t, kr1, br1_t, kavg, br2_row)
    return y[:, :N][None]
